# baseline (speedup 1.0000x reference)
import functools

import jax
import jax.numpy as jnp
from jax import lax
from jax.experimental import pallas as pl
from jax.experimental.pallas import tpu as pltpu
from jax.experimental.pallas import tpu_sc as plsc

_NC = 2
_NS = 16
_TILES = _NC * _NS
_DEGW = 16
_G = 128


def _inv_sqrt(d):
    safe = jnp.where(d > 0, d, 1.0)
    return jnp.where(d > 0, lax.rsqrt(safe), 0.0)


def _mesh():
    return plsc.VectorSubcoreMesh(core_axis_name="c", subcore_axis_name="s",
                                  num_cores=_NC, num_subcores=_NS)


def _make_agg_kernel(npad, d, ng):
    rows_per_tile = npad // _NS
    zchunk = _G
    nz = rows_per_tile // zchunk

    cg = 16
    assert ng % cg == 0

    @functools.partial(
        pl.kernel,
        mesh=_mesh(),
        out_type=jax.ShapeDtypeStruct((_NC, npad, d), jnp.float32),
        scratch_types=[
            pltpu.VMEM_SHARED((npad, d), jnp.float32),
            pltpu.VMEM((cg, _G), jnp.int32),
            pltpu.VMEM((cg, _G), jnp.int32),
            pltpu.VMEM((_G, d), jnp.float32),
            pltpu.VMEM((_G, d), jnp.float32),
            pltpu.SemaphoreType.DMA,
            pltpu.SemaphoreType.DMA,
            pltpu.SemaphoreType.DMA,
            pltpu.SemaphoreType.DMA,
        ],
    )
    def agg_kernel(z_hbm, src_hbm, dst_hbm, zeros_hbm, out_hbm,
                   acc, idx_s, idx_d, rows_a, rows_b,
                   gsem_a, gsem_b, ssem_a, ssem_b):
        c = lax.axis_index("c")
        s = lax.axis_index("s")
        t = c * _NS + s
        base = s * rows_per_tile
        pltpu.sync_copy(zeros_hbm, acc.at[pl.ds(base, rows_per_tile)])
        plsc.subcore_barrier()

        bufs = (rows_a, rows_b)
        gsems = (gsem_a, gsem_b)
        ssems = (ssem_a, ssem_b)

        def chunk_body(cc, carry):
            pltpu.sync_copy(src_hbm.at[t, pl.ds(cc * cg, cg)], idx_s)
            pltpu.sync_copy(dst_hbm.at[t, pl.ds(cc * cg, cg)], idx_d)
            pend_g = [pltpu.async_copy(z_hbm.at[idx_s.at[0]], bufs[0],
                                       gsems[0]), None]
            pend_s = [None, None]
            for j in range(cg):
                p = j % 2
                q = 1 - p
                pend_g[p].wait()
                if pend_s[q] is not None:
                    pend_s[q].wait()
                if j + 1 < cg:
                    pend_g[q] = pltpu.async_copy(z_hbm.at[idx_s.at[j + 1]],
                                                 bufs[q], gsems[q])
                pend_s[p] = pltpu.async_copy(bufs[p], acc.at[idx_d.at[j]],
                                             ssems[p], add=True)
            pend_s[(cg - 1) % 2].wait()
            return carry

        lax.fori_loop(0, ng // cg, chunk_body, 0)
        plsc.subcore_barrier()
        sl = pl.ds(base, rows_per_tile)
        pltpu.sync_copy(acc.at[sl], out_hbm.at[c, sl])

    return agg_kernel


def _make_deg_kernel(npad, ng):
    rows_per_tile = npad // _NS
    zchunk = _G
    nz = rows_per_tile // zchunk
    cg = 16
    assert ng % cg == 0

    @functools.partial(
        pl.kernel,
        mesh=_mesh(),
        out_type=jax.ShapeDtypeStruct((_NC, 2, npad, _DEGW), jnp.float32),
        compiler_params=pltpu.CompilerParams(use_tc_tiling_on_sc=False),
        scratch_types=[
            pltpu.VMEM_SHARED((npad, _DEGW), jnp.float32),
            pltpu.VMEM_SHARED((npad, _DEGW), jnp.float32),
            pltpu.VMEM((cg, _G), jnp.int32),
            pltpu.VMEM((cg, _G), jnp.int32),
            pltpu.VMEM((_G, _DEGW), jnp.float32),
            pltpu.VMEM((_G, _DEGW), jnp.float32),
            pltpu.SemaphoreType.DMA,
        ],
    )
    def deg_kernel(src_hbm, dst_hbm, ones_hbm, zeros_hbm, out_hbm,
                   acc_o, acc_i, idx_s, idx_d, ones_v, stage, sem):
        c = lax.axis_index("c")
        s = lax.axis_index("s")
        t = c * _NS + s
        base = s * rows_per_tile
        pltpu.sync_copy(ones_hbm, ones_v)
        pltpu.sync_copy(zeros_hbm, stage)
        for k in range(nz):
            pltpu.sync_copy(stage, acc_o.at[pl.ds(base + k * zchunk, zchunk)])
            pltpu.sync_copy(stage, acc_i.at[pl.ds(base + k * zchunk, zchunk)])
        plsc.subcore_barrier()

        def chunk_body(cc, carry):
            pltpu.sync_copy(src_hbm.at[t, pl.ds(cc * cg, cg)], idx_s)
            pltpu.sync_copy(dst_hbm.at[t, pl.ds(cc * cg, cg)], idx_d)
            pends = []
            for j in range(cg):
                pends.append(pltpu.async_copy(
                    ones_v, acc_o.at[idx_s.at[j]], sem, add=True))
                pends.append(pltpu.async_copy(
                    ones_v, acc_i.at[idx_d.at[j]], sem, add=True))
            for p in pends:
                p.wait()
            return carry

        lax.fori_loop(0, ng // cg, chunk_body, 0)
        plsc.subcore_barrier()
        for k in range(nz):
            sl = pl.ds(base + k * zchunk, zchunk)
            pltpu.sync_copy(acc_o.at[sl], stage)
            pltpu.sync_copy(stage, out_hbm.at[c, 0, sl])
            pltpu.sync_copy(acc_i.at[sl], stage)
            pltpu.sync_copy(stage, out_hbm.at[c, 1, sl])

    return deg_kernel


def _mm_first_body(x_ref, w_ref, dego_ref, o_ref):
    z = jnp.dot(x_ref[...], w_ref[...], preferred_element_type=jnp.float32)
    o_ref[...] = z * _inv_sqrt(dego_ref[...])


def _mm_mid_body(p_ref, degi_ref, b_ref, w_ref, dego_ref, o_ref):
    h = (p_ref[0] + p_ref[1]) * _inv_sqrt(degi_ref[...]) + b_ref[...]
    h = jnp.maximum(h, 0.0)
    z = jnp.dot(h, w_ref[...], preferred_element_type=jnp.float32)
    o_ref[...] = z * _inv_sqrt(dego_ref[...])


def _make_readout_body(n_real):
    def _readout_body(p_ref, degi_ref, b_ref, wp_ref, bp_ref, o_ref):
        npad = p_ref.shape[1]
        h = (p_ref[0] + p_ref[1]) * _inv_sqrt(degi_ref[...]) + b_ref[...]
        h = jnp.maximum(h, 0.0)
        row = lax.broadcasted_iota(jnp.int32, (npad, 1), 0)
        h = jnp.where(row < n_real, h, 0.0)
        r = jnp.sum(h, axis=0, keepdims=True) * (1.0 / n_real)
        o_ref[...] = jnp.dot(r, wp_ref[...],
                             preferred_element_type=jnp.float32) + bp_ref[...]
    return _readout_body


def kernel(features, edge_index, W1, b1, W2, b2, W3, b3, Wp, bp):
    n, d = features.shape
    e = edge_index.shape[1]
    npad = ((n + _NS * 128 - 1) // (_NS * 128)) * (_NS * 128)
    ept = e // _TILES
    assert ept * _TILES == e
    ng = ((ept + _G - 1) // _G + 7) // 8 * 8
    slots = ng * _G
    padcnt = slots - ept

    src2 = edge_index[0].reshape(_TILES, ept)
    dst2 = edge_index[1].reshape(_TILES, ept)
    padv = (n + (jnp.arange(padcnt, dtype=jnp.int32) % (npad - n)))
    padv = jnp.broadcast_to(padv, (_TILES, padcnt))
    src3 = jnp.concatenate([src2, padv], axis=1).reshape(_TILES, ng, _G)
    dst3 = jnp.concatenate([dst2, padv], axis=1).reshape(_TILES, ng, _G)

    zerosd = jnp.zeros((npad // _NS, d), jnp.float32)
    onesd = jnp.ones((_G, _DEGW), jnp.float32)
    zerosw = jnp.zeros((_G, _DEGW), jnp.float32)
    xpad = jnp.zeros((npad, d), features.dtype).at[:n].set(features)

    agg = _make_agg_kernel(npad, d, ng)
    deg = _make_deg_kernel(npad, ng)

    degp = deg(src3, dst3, onesd, zerosw)
    dego = degp[0, 0, :, :1] + degp[1, 0, :, :1]
    degi = degp[0, 1, :, :1] + degp[1, 1, :, :1]
    mm_first = pl.pallas_call(
        _mm_first_body, out_shape=jax.ShapeDtypeStruct((npad, d), jnp.float32))
    mm_mid = pl.pallas_call(
        _mm_mid_body, out_shape=jax.ShapeDtypeStruct((npad, d), jnp.float32))
    readout = pl.pallas_call(
        _make_readout_body(n),
        out_shape=jax.ShapeDtypeStruct((1, Wp.shape[1]), jnp.float32))

    b1r, b2r, b3r = b1.reshape(1, d), b2.reshape(1, d), b3.reshape(1, d)
    bpr = bp.reshape(1, -1)

    z1 = mm_first(xpad, W1, dego)
    p1 = agg(z1, src3, dst3, zerosd)
    z2 = mm_mid(p1, degi, b1r, W2, dego)
    p2 = agg(z2, src3, dst3, zerosd)
    z3 = mm_mid(p2, degi, b2r, W3, dego)
    p3 = agg(z3, src3, dst3, zerosd)
    return readout(p3, degi, b3r, Wp, bpr)


# speedup vs baseline: 1.0031x; 1.0031x over previous
import functools

import jax
import jax.numpy as jnp
from jax import lax
from jax.experimental import pallas as pl
from jax.experimental.pallas import tpu as pltpu
from jax.experimental.pallas import tpu_sc as plsc

_NC = 2
_NS = 16
_TILES = _NC * _NS
_DEGW = 16
_G = 128


def _inv_sqrt(d):
    safe = jnp.where(d > 0, d, 1.0)
    return jnp.where(d > 0, 1.0 / jnp.sqrt(safe), 0.0)


def _mesh():
    return plsc.VectorSubcoreMesh(core_axis_name="c", subcore_axis_name="s",
                                  num_cores=_NC, num_subcores=_NS)


def _make_agg_kernel(npad, d, ng):
    rows_per_tile = npad // _NS
    zchunk = _G
    nz = rows_per_tile // zchunk

    cg = 16
    assert ng % cg == 0

    @functools.partial(
        pl.kernel,
        mesh=_mesh(),
        out_type=jax.ShapeDtypeStruct((_NC, npad, d), jnp.float32),
        scratch_types=[
            pltpu.VMEM_SHARED((npad, d), jnp.float32),
            pltpu.VMEM((cg, _G), jnp.int32),
            pltpu.VMEM((cg, _G), jnp.int32),
            pltpu.VMEM((_G, d), jnp.float32),
            pltpu.VMEM((_G, d), jnp.float32),
            pltpu.SemaphoreType.DMA,
            pltpu.SemaphoreType.DMA,
            pltpu.SemaphoreType.DMA,
            pltpu.SemaphoreType.DMA,
        ],
    )
    def agg_kernel(z_hbm, src_hbm, dst_hbm, zeros_hbm, out_hbm,
                   acc, idx_s, idx_d, rows_a, rows_b,
                   gsem_a, gsem_b, ssem_a, ssem_b):
        c = lax.axis_index("c")
        s = lax.axis_index("s")
        t = c * _NS + s
        base = s * rows_per_tile
        pltpu.sync_copy(zeros_hbm, rows_a)
        for k in range(nz):
            pltpu.sync_copy(rows_a, acc.at[pl.ds(base + k * zchunk, zchunk)])
        plsc.subcore_barrier()

        bufs = (rows_a, rows_b)
        gsems = (gsem_a, gsem_b)
        ssems = (ssem_a, ssem_b)

        def chunk_body(cc, carry):
            pltpu.sync_copy(src_hbm.at[t, pl.ds(cc * cg, cg)], idx_s)
            pltpu.sync_copy(dst_hbm.at[t, pl.ds(cc * cg, cg)], idx_d)
            pend_g = [pltpu.async_copy(z_hbm.at[idx_s.at[0]], bufs[0],
                                       gsems[0]), None]
            pend_s = [None, None]
            for j in range(cg):
                p = j % 2
                q = 1 - p
                pend_g[p].wait()
                if pend_s[q] is not None:
                    pend_s[q].wait()
                if j + 1 < cg:
                    pend_g[q] = pltpu.async_copy(z_hbm.at[idx_s.at[j + 1]],
                                                 bufs[q], gsems[q])
                pend_s[p] = pltpu.async_copy(bufs[p], acc.at[idx_d.at[j]],
                                             ssems[p], add=True)
            pend_s[(cg - 1) % 2].wait()
            return carry

        lax.fori_loop(0, ng // cg, chunk_body, 0)
        plsc.subcore_barrier()
        for k in range(nz):
            sl = pl.ds(base + k * zchunk, zchunk)
            pltpu.sync_copy(acc.at[sl], rows_a)
            pltpu.sync_copy(rows_a, out_hbm.at[c, sl])

    return agg_kernel


def _make_deg_kernel(npad, ng):
    rows_per_tile = npad // _NS
    zchunk = _G
    nz = rows_per_tile // zchunk
    cg = 16
    assert ng % cg == 0

    @functools.partial(
        pl.kernel,
        mesh=_mesh(),
        out_type=jax.ShapeDtypeStruct((_NC, 2, npad, _DEGW), jnp.float32),
        compiler_params=pltpu.CompilerParams(use_tc_tiling_on_sc=False),
        scratch_types=[
            pltpu.VMEM_SHARED((npad, _DEGW), jnp.float32),
            pltpu.VMEM_SHARED((npad, _DEGW), jnp.float32),
            pltpu.VMEM((cg, _G), jnp.int32),
            pltpu.VMEM((cg, _G), jnp.int32),
            pltpu.VMEM((_G, _DEGW), jnp.float32),
            pltpu.VMEM((_G, _DEGW), jnp.float32),
            pltpu.SemaphoreType.DMA,
        ],
    )
    def deg_kernel(src_hbm, dst_hbm, ones_hbm, zeros_hbm, out_hbm,
                   acc_o, acc_i, idx_s, idx_d, ones_v, stage, sem):
        c = lax.axis_index("c")
        s = lax.axis_index("s")
        t = c * _NS + s
        base = s * rows_per_tile
        pltpu.sync_copy(ones_hbm, ones_v)
        pltpu.sync_copy(zeros_hbm, stage)
        for k in range(nz):
            pltpu.sync_copy(stage, acc_o.at[pl.ds(base + k * zchunk, zchunk)])
            pltpu.sync_copy(stage, acc_i.at[pl.ds(base + k * zchunk, zchunk)])
        plsc.subcore_barrier()

        def chunk_body(cc, carry):
            pltpu.sync_copy(src_hbm.at[t, pl.ds(cc * cg, cg)], idx_s)
            pltpu.sync_copy(dst_hbm.at[t, pl.ds(cc * cg, cg)], idx_d)
            pends = []
            for j in range(cg):
                pends.append(pltpu.async_copy(
                    ones_v, acc_o.at[idx_s.at[j]], sem, add=True))
                pends.append(pltpu.async_copy(
                    ones_v, acc_i.at[idx_d.at[j]], sem, add=True))
            for p in pends:
                p.wait()
            return carry

        lax.fori_loop(0, ng // cg, chunk_body, 0)
        plsc.subcore_barrier()
        for k in range(nz):
            sl = pl.ds(base + k * zchunk, zchunk)
            pltpu.sync_copy(acc_o.at[sl], stage)
            pltpu.sync_copy(stage, out_hbm.at[c, 0, sl])
            pltpu.sync_copy(acc_i.at[sl], stage)
            pltpu.sync_copy(stage, out_hbm.at[c, 1, sl])

    return deg_kernel


def _scale_body(x_ref, dego_ref, o_ref):
    o_ref[...] = x_ref[...] * _inv_sqrt(dego_ref[...])


def _mm_mid_body(p_ref, degi_ref, w_ref, b_ref, dego_ref, o_ref):
    agg = (p_ref[0] + p_ref[1]) * _inv_sqrt(degi_ref[...])
    h = jnp.dot(agg, w_ref[...], preferred_element_type=jnp.float32) + b_ref[...]
    h = jnp.maximum(h, 0.0)
    o_ref[...] = h * _inv_sqrt(dego_ref[...])


def _make_readout_body(n_real):
    def _readout_body(p_ref, degi_ref, w_ref, b_ref, wp_ref, bp_ref, o_ref):
        npad = p_ref.shape[1]
        agg = (p_ref[0] + p_ref[1]) * _inv_sqrt(degi_ref[...])
        h = jnp.dot(agg, w_ref[...],
                    preferred_element_type=jnp.float32) + b_ref[...]
        h = jnp.maximum(h, 0.0)
        row = lax.broadcasted_iota(jnp.int32, (npad, 1), 0)
        h = jnp.where(row < n_real, h, 0.0)
        r = jnp.sum(h, axis=0, keepdims=True) * (1.0 / n_real)
        o_ref[...] = (jnp.sum(r * wp_ref[...], axis=1, keepdims=True)
                      + bp_ref[...])
    return _readout_body


def kernel(features, edge_index, W1, b1, W2, b2, W3, b3, Wp, bp):
    n, d = features.shape
    e = edge_index.shape[1]
    npad = ((n + _NS * 128 - 1) // (_NS * 128)) * (_NS * 128)
    ept = e // _TILES
    assert ept * _TILES == e
    ng = ((ept + _G - 1) // _G + 7) // 8 * 8
    slots = ng * _G
    padcnt = slots - ept

    src2 = edge_index[0].reshape(_TILES, ept)
    dst2 = edge_index[1].reshape(_TILES, ept)
    padv = (n + (jnp.arange(padcnt, dtype=jnp.int32) % (npad - n)))
    padv = jnp.broadcast_to(padv, (_TILES, padcnt))
    src3 = jnp.concatenate([src2, padv], axis=1).reshape(_TILES, ng, _G)
    dst3 = jnp.concatenate([dst2, padv], axis=1).reshape(_TILES, ng, _G)

    zerosd = jnp.zeros((_G, d), jnp.float32)
    onesd = jnp.ones((_G, _DEGW), jnp.float32)
    zerosw = jnp.zeros((_G, _DEGW), jnp.float32)
    xpad = jnp.zeros((npad, d), features.dtype).at[:n].set(features)

    agg = _make_agg_kernel(npad, d, ng)
    deg = _make_deg_kernel(npad, ng)

    degp = deg(src3, dst3, onesd, zerosw)
    dego = degp[0, 0, :, :1] + degp[1, 0, :, :1]
    degi = degp[0, 1, :, :1] + degp[1, 1, :, :1]
    scale = pl.pallas_call(
        _scale_body, out_shape=jax.ShapeDtypeStruct((npad, d), jnp.float32))
    mm_mid = pl.pallas_call(
        _mm_mid_body, out_shape=jax.ShapeDtypeStruct((npad, d), jnp.float32))
    readout = pl.pallas_call(
        _make_readout_body(n),
        out_shape=jax.ShapeDtypeStruct((1, Wp.shape[1]), jnp.float32))

    b1r, b2r, b3r = b1.reshape(1, d), b2.reshape(1, d), b3.reshape(1, d)
    bpr = bp.reshape(1, -1)

    x1 = scale(xpad, dego)
    p1 = agg(x1, src3, dst3, zerosd)
    h1 = mm_mid(p1, degi, W1, b1r, dego)
    p2 = agg(h1, src3, dst3, zerosd)
    h2 = mm_mid(p2, degi, W2, b2r, dego)
    p3 = agg(h2, src3, dst3, zerosd)
    return readout(p3, degi, W3, b3r, Wp.reshape(1, -1), bpr)
